# baseline (speedup 1.0000x reference)
import functools

import jax
import jax.numpy as jnp
from jax import lax
from jax.experimental import pallas as pl
from jax.experimental.pallas import tpu as pltpu
from jax.experimental.pallas import tpu_sc as plsc

_HIDDEN = 64
_NB, _NS_SEQ = 16384, 50
_NC, _NT = 2, 16
_NW = _NC * _NT
_SB = _NB // _NW
_BLK = 128
_NBLK = _SB // _BLK
_L = 16


def _gather_sc(ids_flat, table):
  mesh = plsc.VectorSubcoreMesh(core_axis_name="c", subcore_axis_name="s")

  @functools.partial(
      pl.kernel,
      out_type=jax.ShapeDtypeStruct((_NS_SEQ, 8, _NB // _BLK, 8 * _BLK),
                                    jnp.float32),
      mesh=mesh,
      compiler_params=pltpu.CompilerParams(use_tc_tiling_on_sc=False,
                                           needs_layout_passes=False),
      scratch_types=(
          [pltpu.VMEM((_SB * _NS_SEQ,), jnp.int32),
           pltpu.VMEM((_NS_SEQ, _NBLK, _BLK), jnp.int32)]
          + [pltpu.VMEM((_BLK, _HIDDEN), jnp.float32) for _ in range(_NBLK)]
          + [pltpu.VMEM((8, 8 * _BLK), jnp.float32) for _ in range(_NBLK)]
          + [pltpu.SemaphoreType.DMA for _ in range(2 * _NBLK + 1)]
      ),
  )
  def body(ids_hbm, table_hbm, out_hbm, ids_v, idst, *rest):
    rbuf = rest[:_NBLK]
    tbuf = rest[_NBLK:2 * _NBLK]
    gsem = rest[2 * _NBLK:3 * _NBLK]
    osem = rest[3 * _NBLK:4 * _NBLK]
    isem = rest[4 * _NBLK]
    wid = lax.axis_index("s") * _NC + lax.axis_index("c")

    iota = lax.iota(jnp.int32, _L)
    zeros = iota * 0
    pltpu.async_copy(ids_hbm.at[pl.ds(wid * (_SB * _NS_SEQ), _SB * _NS_SEQ)],
                     ids_v, isem).wait()

    rows_a = [[(iota + (blk * _BLK + q * _L)) * _NS_SEQ
               for q in range(_BLK // _L)] for blk in range(_NBLK)]

    @pl.loop(0, _NS_SEQ, init_carry=zeros)
    def _tr_ids(s, svec):
      for blk in range(_NBLK):
        for q in range(_BLK // _L):
          vals = plsc.load_gather(ids_v, [rows_a[blk][q] + svec])
          idst[s, blk, pl.ds(q * _L, _L)] = vals
      return svec + 1

    def fire_gather(blk, s):
      pltpu.async_copy(table_hbm.at[idst.at[s, blk]], rbuf[blk], gsem[blk])

    def wait_gather(blk):
      pltpu.make_async_copy(table_hbm.at[idst.at[0, blk]], rbuf[blk],
                            gsem[blk]).wait()

    hr_a = [(iota + 16 * c) // 8 for c in range(_HIDDEN // _L)]
    i1_a = [((iota + 16 * c) % 8) * _BLK for c in range(_HIDDEN // _L)]

    def unit(s, s2, blk, first):
      wait_gather(blk)
      if not first:
        pltpu.make_async_copy(tbuf[blk], out_hbm.at[0, slice(None), 0],
                              osem[blk]).wait()

      @pl.loop(0, _BLK, init_carry=zeros, unroll=4)
      def _tr(j, jvec):
        for c in range(_HIDDEN // _L):
          x = rbuf[blk][j, pl.ds(c * _L, _L)]
          plsc.store_scatter(tbuf[blk], [hr_a[c], i1_a[c] + jvec], x)
        return jvec + 1

      bc = wid * _NBLK + blk
      pltpu.async_copy(tbuf[blk], out_hbm.at[s, slice(None), bc],
                       osem[blk])
      fire_gather(blk, s2)

    for blk in range(_NBLK):
      fire_gather(blk, 0)
    for blk in range(_NBLK):
      unit(0, 1, blk, True)

    @pl.loop(1, _NS_SEQ)
    def _units(s):
      s2 = jnp.minimum(s + 1, _NS_SEQ - 1)
      for blk in range(_NBLK):
        unit(s, s2, blk, False)

    for blk in range(_NBLK):
      wait_gather(blk)
      pltpu.make_async_copy(tbuf[blk], out_hbm.at[0, slice(None), 0],
                            osem[blk]).wait()

  return body(ids_flat, table)


def kernel(tool_ids, table):
  out4d = _gather_sc(tool_ids.astype(jnp.int32).reshape(-1), table)
  return (out4d.reshape(_NS_SEQ, 8, _NB // _BLK, 8, _BLK)
          .transpose(2, 4, 0, 1, 3).reshape(_NB, _NS_SEQ, _HIDDEN))


# speedup vs baseline: 1.0743x; 1.0743x over previous
import functools

import jax
import jax.numpy as jnp
from jax import lax
from jax.experimental import pallas as pl
from jax.experimental.pallas import tpu as pltpu
from jax.experimental.pallas import tpu_sc as plsc

_HIDDEN = 64
_NB, _NS_SEQ = 16384, 50
_NC, _NT = 2, 16
_NW = _NC * _NT
_SB = _NB // _NW
_BLK = 128
_NBLK = _SB // _BLK
_L = 16


def _gather_sc(ids_flat, table):
  mesh = plsc.VectorSubcoreMesh(core_axis_name="c", subcore_axis_name="s")

  @functools.partial(
      pl.kernel,
      out_type=jax.ShapeDtypeStruct((_NS_SEQ, 8, _NB // _BLK, 8 * _BLK),
                                    jnp.float32),
      mesh=mesh,
      compiler_params=pltpu.CompilerParams(use_tc_tiling_on_sc=False,
                                           needs_layout_passes=False),
      scratch_types=(
          [pltpu.VMEM((_SB * _NS_SEQ,), jnp.int32),
           pltpu.VMEM((_NS_SEQ, _NBLK, _BLK), jnp.int32)]
          + [pltpu.VMEM((_BLK, _HIDDEN), jnp.float32) for _ in range(_NBLK)]
          + [pltpu.VMEM((8, 8 * _BLK), jnp.float32) for _ in range(_NBLK)]
          + [pltpu.SemaphoreType.DMA for _ in range(2 * _NBLK + 1)]
      ),
  )
  def body(ids_hbm, table_hbm, out_hbm, ids_v, idst, *rest):
    rbuf = rest[:_NBLK]
    tbuf = rest[_NBLK:2 * _NBLK]
    gsem = rest[2 * _NBLK:3 * _NBLK]
    osem = rest[3 * _NBLK:4 * _NBLK]
    isem = rest[4 * _NBLK]
    wid = lax.axis_index("s") * _NC + lax.axis_index("c")

    iota = lax.iota(jnp.int32, _L)
    zeros = iota * 0
    pltpu.async_copy(ids_hbm.at[pl.ds(wid * (_SB * _NS_SEQ), _SB * _NS_SEQ)],
                     ids_v, isem).wait()

    rows_a = [[(iota + (blk * _BLK + q * _L)) * _NS_SEQ
               for q in range(_BLK // _L)] for blk in range(_NBLK)]

    @pl.loop(0, _NS_SEQ, init_carry=zeros)
    def _tr_ids(s, svec):
      for blk in range(_NBLK):
        for q in range(_BLK // _L):
          vals = plsc.load_gather(ids_v, [rows_a[blk][q] + svec])
          idst[s, blk, pl.ds(q * _L, _L)] = vals + vals
      return svec + 1

    def fire_gather(blk, s):
      pltpu.async_copy(table_hbm.at[idst.at[s, blk]], rbuf[blk], gsem[blk])

    def wait_gather(blk):
      pltpu.make_async_copy(table_hbm.at[idst.at[0, blk]], rbuf[blk],
                            gsem[blk]).wait()

    _NCH = _HIDDEN // _L
    hr_a = [(iota + 16 * c) // 8 for c in range(_NCH)]
    i1_a = [[((iota + 16 * c) % 8) * _BLK + dj for c in range(_NCH)]
            for dj in range(4)]

    def unit(s, s2, blk, first):
      wait_gather(blk)
      if not first:
        pltpu.make_async_copy(tbuf[blk], out_hbm.at[0, slice(None), 0],
                              osem[blk]).wait()

      @pl.loop(0, _BLK, step=4, init_carry=zeros)
      def _tr(j, jvec):
        xs = [rbuf[blk][j + dj, pl.ds(c * _L, _L)]
              for dj in range(4) for c in range(_NCH)]
        k = 0
        for dj in range(4):
          for c in range(_NCH):
            plsc.store_scatter(tbuf[blk], [hr_a[c], i1_a[dj][c] + jvec],
                               xs[k])
            k += 1
        return jvec + 4

      bc = wid * _NBLK + blk
      pltpu.async_copy(tbuf[blk], out_hbm.at[s, slice(None), bc],
                       osem[blk])
      fire_gather(blk, s2)

    for blk in range(_NBLK):
      fire_gather(blk, 0)
    for blk in range(_NBLK):
      unit(0, 1, blk, True)

    @pl.loop(1, _NS_SEQ)
    def _units(s):
      s2 = jnp.minimum(s + 1, _NS_SEQ - 1)
      for blk in range(_NBLK):
        unit(s, s2, blk, False)

    for blk in range(_NBLK):
      wait_gather(blk)
      pltpu.make_async_copy(tbuf[blk], out_hbm.at[0, slice(None), 0],
                            osem[blk]).wait()

  return body(ids_flat, table)


def kernel(tool_ids, table):
  table_pad = jnp.pad(table, ((0, 0), (0, 64))).reshape(2 * 1000000, 64)
  out4d = _gather_sc(tool_ids.astype(jnp.int32).reshape(-1), table_pad)
  return (out4d.reshape(_NS_SEQ, 8, _NB // _BLK, 8, _BLK)
          .transpose(2, 4, 0, 1, 3).reshape(_NB, _NS_SEQ, _HIDDEN))


# speedup vs baseline: 1.9621x; 1.8263x over previous
import functools

import jax
import jax.numpy as jnp
from jax import lax
from jax.experimental import pallas as pl
from jax.experimental.pallas import tpu as pltpu
from jax.experimental.pallas import tpu_sc as plsc

_HIDDEN = 64
_NB, _NS_SEQ = 16384, 50
_NC, _NT = 2, 16
_NW = _NC * _NT
_SB = _NB // _NW
_BLK = 128
_NBLK = _SB // _BLK
_L = 16


def _gather_sc(ids_flat, table):
  mesh = plsc.VectorSubcoreMesh(core_axis_name="c", subcore_axis_name="s")

  @functools.partial(
      pl.kernel,
      out_type=jax.ShapeDtypeStruct((_NS_SEQ, 8, _NB // _BLK, 8, _BLK),
                                    jnp.float32),
      mesh=mesh,
      compiler_params=pltpu.CompilerParams(use_tc_tiling_on_sc=False,
                                           needs_layout_passes=False),
      scratch_types=(
          [pltpu.VMEM((_SB * _NS_SEQ,), jnp.int32),
           pltpu.VMEM((_NS_SEQ, _NBLK, _BLK), jnp.int32)]
          + [pltpu.VMEM((_BLK, _HIDDEN), jnp.float32) for _ in range(_NBLK)]
          + [pltpu.VMEM((8, 8, _BLK + 1), jnp.float32) for _ in range(_NBLK)]
          + [pltpu.SemaphoreType.DMA for _ in range(2 * _NBLK + 1)]
      ),
  )
  def body(ids_hbm, table_hbm, out_hbm, ids_v, idst, *rest):
    rbuf = rest[:_NBLK]
    tbuf = rest[_NBLK:2 * _NBLK]
    gsem = rest[2 * _NBLK:3 * _NBLK]
    osem = rest[3 * _NBLK:4 * _NBLK]
    isem = rest[4 * _NBLK]
    wid = lax.axis_index("s") * _NC + lax.axis_index("c")

    iota = lax.iota(jnp.int32, _L)
    zeros = iota * 0
    pltpu.async_copy(ids_hbm.at[pl.ds(wid * (_SB * _NS_SEQ), _SB * _NS_SEQ)],
                     ids_v, isem).wait()

    rows_a = [[(iota + (blk * _BLK + q * _L)) * _NS_SEQ
               for q in range(_BLK // _L)] for blk in range(_NBLK)]

    @pl.loop(0, _NS_SEQ, init_carry=zeros)
    def _tr_ids(s, svec):
      for blk in range(_NBLK):
        for q in range(_BLK // _L):
          vals = plsc.load_gather(ids_v, [rows_a[blk][q] + svec])
          idst[s, blk, pl.ds(q * _L, _L)] = vals + vals
      return svec + 1

    def fire_gather(blk, s):
      pltpu.async_copy(table_hbm.at[idst.at[s, blk]], rbuf[blk], gsem[blk])

    def wait_gather(blk):
      pltpu.make_async_copy(table_hbm.at[idst.at[0, blk]], rbuf[blk],
                            gsem[blk]).wait()

    _NCH = _HIDDEN // _L
    hr_a = [(iota + 16 * c) // 8 for c in range(_NCH)]
    hi_a = [(iota + 16 * c) % 8 for c in range(_NCH)]
    tsl = (slice(None), slice(None), pl.ds(0, _BLK))

    def unit(s, s2, blk, first):
      wait_gather(blk)
      if not first:
        pltpu.make_async_copy(tbuf[blk].at[tsl],
                              out_hbm.at[0, slice(None), 0],
                              osem[blk]).wait()

      @pl.loop(0, _BLK, step=4, init_carry=zeros)
      def _tr(j, jvec):
        xs = [rbuf[blk][j + dj, pl.ds(c * _L, _L)]
              for dj in range(4) for c in range(_NCH)]
        k = 0
        for dj in range(4):
          for c in range(_NCH):
            plsc.store_scatter(tbuf[blk],
                               [hr_a[c], hi_a[c], jvec + dj], xs[k])
            k += 1
        return jvec + 4

      bc = wid * _NBLK + blk
      pltpu.async_copy(tbuf[blk].at[tsl], out_hbm.at[s, slice(None), bc],
                       osem[blk])
      fire_gather(blk, s2)

    for blk in range(_NBLK):
      fire_gather(blk, 0)
    for blk in range(_NBLK):
      unit(0, 1, blk, True)

    @pl.loop(1, _NS_SEQ)
    def _units(s):
      s2 = jnp.minimum(s + 1, _NS_SEQ - 1)
      for blk in range(_NBLK):
        unit(s, s2, blk, False)

    for blk in range(_NBLK):
      wait_gather(blk)
      pltpu.make_async_copy(tbuf[blk].at[tsl],
                            out_hbm.at[0, slice(None), 0],
                            osem[blk]).wait()

  return body(ids_flat, table)


def kernel(tool_ids, table):
  table_pad = jnp.pad(table, ((0, 0), (0, 64))).reshape(2 * 1000000, 64)
  out5d = _gather_sc(tool_ids.astype(jnp.int32).reshape(-1), table_pad)
  return out5d.transpose(2, 4, 0, 1, 3).reshape(_NB, _NS_SEQ, _HIDDEN)
